# baseline (speedup 1.0000x reference)
import jax
import jax.numpy as jnp
from jax import lax
from jax.experimental import pallas as pl
from jax.experimental.pallas import tpu as pltpu
from jax.experimental.pallas import tpu_sc as plsc

TOT = 128
D = 4096
BATCH = 128
B = BATCH * TOT

_info = plsc.get_sparse_core_info()
NC, NS = _info.num_cores, _info.num_subcores
NW = NC * NS
B_PER_W = B // NW
C = 32
G = B_PER_W // C


def _body(idx_hbm, table_hbm, out_hbm, idx_v, table_sp, sbuf, rbuf,
          dsem, gsem, ssem, rgsem, rssem):
    sid = lax.axis_index("s")
    wid = sid * NC + lax.axis_index("c")
    base = wid * B_PER_W
    pltpu.sync_copy(idx_hbm.at[wid], idx_v)
    rpw = TOT // NS
    pltpu.sync_copy(table_hbm.at[pl.ds(sid * rpw, rpw)],
                    table_sp.at[pl.ds(sid * rpw, rpw)])
    plsc.subcore_barrier()

    def s_write_desc(h):
        return pltpu.make_async_copy(
            sbuf, out_hbm.at[pl.ds(base + h * C, 8)], ssem)

    def r_write_desc(h):
        return pltpu.make_async_copy(
            rbuf, out_hbm.at[pl.ds(base + h * C + 24, 8)], rssem)

    def step(h, carry):
        @pl.when(h >= 1)
        def _():
            s_write_desc(h - 1).wait()
            r_write_desc(h - 1).wait()

        vec0 = idx_v[pl.ds(h * C, 16)]
        vec1 = idx_v[pl.ds(h * C + 16, 16)]

        for jj in range(8):
            pltpu.async_copy(table_sp.at[pl.ds(vec0[jj], 1)],
                             sbuf.at[pl.ds(jj, 1)], gsem)
        pltpu.async_copy(table_hbm.at[idx_v.at[pl.ds(h * C + 24, 8)]],
                         rbuf, rgsem)
        for jj in range(8):
            pltpu.async_copy(table_sp.at[pl.ds(vec0[8 + jj], 1)],
                             out_hbm.at[pl.ds(base + h * C + 8 + jj, 1)],
                             dsem)
        for jj in range(8):
            pltpu.async_copy(table_sp.at[pl.ds(vec1[jj], 1)],
                             out_hbm.at[pl.ds(base + h * C + 16 + jj, 1)],
                             dsem)

        for jj in range(8):
            pltpu.make_async_copy(table_sp.at[pl.ds(0, 1)],
                                  sbuf.at[pl.ds(jj, 1)], gsem).wait()
        pltpu.async_copy(sbuf, out_hbm.at[pl.ds(base + h * C, 8)], ssem)

        pltpu.make_async_copy(table_hbm.at[idx_v.at[pl.ds(h * C + 24, 8)]],
                              rbuf, rgsem).wait()
        pltpu.async_copy(rbuf, out_hbm.at[pl.ds(base + h * C + 24, 8)],
                         rssem)

        for jj in range(16):
            pltpu.make_async_copy(table_sp.at[pl.ds(0, 1)],
                                  out_hbm.at[pl.ds(base, 1)], dsem).wait()
        return carry

    lax.fori_loop(0, G, step, 0)
    s_write_desc(G - 1).wait()
    r_write_desc(G - 1).wait()


_gather = pl.kernel(
    _body,
    out_type=jax.ShapeDtypeStruct((B, D), jnp.float32),
    mesh=plsc.VectorSubcoreMesh(core_axis_name="c", subcore_axis_name="s"),
    scratch_types=[
        pltpu.VMEM((B_PER_W,), jnp.int32),
        pltpu.VMEM_SHARED((TOT, D), jnp.float32),
        pltpu.VMEM((8, D), jnp.float32),
        pltpu.VMEM((8, D), jnp.float32),
        pltpu.SemaphoreType.DMA,
        pltpu.SemaphoreType.DMA,
        pltpu.SemaphoreType.DMA,
        pltpu.SemaphoreType.DMA,
        pltpu.SemaphoreType.DMA,
    ],
)


def kernel(indices, embedding):
    idx = indices.astype(jnp.int32).reshape(NW, B_PER_W)
    out = _gather(idx, embedding)
    return out.reshape(BATCH, TOT, D)
